# baseline (speedup 1.0000x reference)
import functools

import jax
import jax.numpy as jnp
from jax.experimental import pallas as pl
from jax.experimental.pallas import tpu as pltpu

EPS = 1e-12
Q = 784
D = 512
K = 16384
KT = 2048


def _knn_kernel(q_ref, m_ref, minval_ref, sidx_ref, star_ref, sstar_ref,
                m2_ref, qs_ref, ming_ref, amin_ref):
    t = pl.program_id(0)
    nt = pl.num_programs(0)

    @pl.when(t == 0)
    def _prescale():
        qs_ref[...] = -2.0 * q_ref[...]

    m = m_ref[...]
    qm = jax.lax.dot_general(
        qs_ref[...], m, (((1,), (1,)), ((), ())),
        preferred_element_type=jnp.float32)
    m2 = jax.lax.dot_general(
        jnp.ones((1, D), jnp.float32), m * m, (((1,), (1,)), ((), ())),
        preferred_element_type=jnp.float32)
    m2_ref[...] = m2[None]
    g = qm + m2
    rowmin = jnp.min(g, axis=1, keepdims=True)
    lanes = jax.lax.broadcasted_iota(jnp.int32, g.shape, 1)
    rowarg = jnp.min(jnp.where(g == rowmin, lanes, K),
                     axis=1, keepdims=True) + t * KT

    @pl.when(t == 0)
    def _init():
        ming_ref[...] = rowmin
        amin_ref[...] = rowarg

    @pl.when(t > 0)
    def _update():
        better = rowmin < ming_ref[...]
        amin_ref[...] = jnp.where(better, rowarg, amin_ref[...])
        ming_ref[...] = jnp.where(better, rowmin, ming_ref[...])

    @pl.when(t == nt - 1)
    def _finalize():
        q = q_ref[...]
        q2 = jnp.sum(q * q, axis=1, keepdims=True)
        mv2 = jnp.maximum(ming_ref[...] + q2, 0.0)
        minval_ref[...] = jnp.sqrt(mv2 + EPS)
        smax = jnp.max(mv2)
        rows = jax.lax.broadcasted_iota(jnp.int32, mv2.shape, 0)
        sidx = jnp.min(jnp.where(mv2 == smax, rows, Q))
        star = jnp.min(jnp.where(rows == sidx, amin_ref[...], K))
        sidx_ref[0, 0] = sidx
        star_ref[0, 0] = star
        sstar_ref[0, 0] = jnp.sqrt(smax + EPS)


def _reweight_kernel(m_ref, m2_ref, ms_ref, sstar_ref, s_ref,
                     wstar_ref, wtest_ref):
    t = pl.program_id(0)
    nt = pl.num_programs(0)
    m = m_ref[...]
    m2 = m2_ref[0]
    cross = jax.lax.dot_general(
        ms_ref[...], m, (((1,), (1,)), ((), ())),
        preferred_element_type=jnp.float32)
    wstar_ref[pl.ds(t, 1), :] = m2 - 2.0 * cross[0:1, :]
    wtest_ref[pl.ds(t, 1), :] = m2 - 2.0 * cross[1:2, :]

    @pl.when(t == nt - 1)
    def _finalize():
        ms = ms_ref[...]
        s2 = jnp.sum(ms[0:1, :] * ms[0:1, :])
        t2 = jnp.sum(ms[1:2, :] * ms[1:2, :])
        ws = jnp.maximum(wstar_ref[...] + s2, 0.0)
        wt = jnp.maximum(wtest_ref[...] + t2, 0.0)
        rows = jax.lax.broadcasted_iota(jnp.int32, ws.shape, 0)
        lanes = jax.lax.broadcasted_iota(jnp.int32, ws.shape, 1)
        flat = rows * KT + lanes
        acc = 0.0
        for _ in range(3):
            mn = jnp.min(ws)
            idx = jnp.min(jnp.where(ws == mn, flat, K))
            dj2 = jnp.min(jnp.where(flat == idx, wt, jnp.inf))
            acc = acc + jnp.exp(jnp.sqrt(dj2 + EPS))
            ws = jnp.where(flat == idx, jnp.inf, ws)
        s_star = sstar_ref[0, 0]
        s_ref[0, 0] = (1.0 - jnp.exp(s_star) / acc) * s_star


@functools.partial(jax.jit, static_argnums=())
def kernel(queries, memory):
    nt = K // KT
    minval, sidx, star, sstar, m2 = pl.pallas_call(
        _knn_kernel,
        grid=(nt,),
        in_specs=[
            pl.BlockSpec((Q, D), lambda t: (0, 0)),
            pl.BlockSpec((KT, D), lambda t: (t, 0)),
        ],
        out_specs=[
            pl.BlockSpec((Q, 1), lambda t: (0, 0)),
            pl.BlockSpec(memory_space=pltpu.SMEM),
            pl.BlockSpec(memory_space=pltpu.SMEM),
            pl.BlockSpec(memory_space=pltpu.SMEM),
            pl.BlockSpec((1, 1, KT), lambda t: (t, 0, 0)),
        ],
        out_shape=[
            jax.ShapeDtypeStruct((Q, 1), jnp.float32),
            jax.ShapeDtypeStruct((1, 1), jnp.int32),
            jax.ShapeDtypeStruct((1, 1), jnp.int32),
            jax.ShapeDtypeStruct((1, 1), jnp.float32),
            jax.ShapeDtypeStruct((nt, 1, KT), jnp.float32),
        ],
        scratch_shapes=[
            pltpu.VMEM((Q, D), jnp.float32),
            pltpu.VMEM((Q, 1), jnp.float32),
            pltpu.VMEM((Q, 1), jnp.int32),
        ],
    )(queries, memory)

    m_star = jnp.take(memory, star[0, 0], axis=0)[None, :]
    m_test = jnp.take(queries, sidx[0, 0], axis=0)[None, :]
    ms = jnp.concatenate([m_star, m_test], axis=0)

    s = pl.pallas_call(
        _reweight_kernel,
        grid=(nt,),
        in_specs=[
            pl.BlockSpec((KT, D), lambda t: (t, 0)),
            pl.BlockSpec((1, 1, KT), lambda t: (t, 0, 0)),
            pl.BlockSpec((2, D), lambda t: (0, 0)),
            pl.BlockSpec(memory_space=pltpu.SMEM),
        ],
        out_specs=pl.BlockSpec(memory_space=pltpu.SMEM),
        out_shape=jax.ShapeDtypeStruct((1, 1), jnp.float32),
        scratch_shapes=[
            pltpu.VMEM((nt, KT), jnp.float32),
            pltpu.VMEM((nt, KT), jnp.float32),
        ],
    )(memory, m2, ms, sstar)

    return (s[0, 0], minval.reshape(Q))


# speedup vs baseline: 1.1497x; 1.1497x over previous
import functools

import jax
import jax.numpy as jnp
from jax.experimental import pallas as pl
from jax.experimental.pallas import tpu as pltpu

EPS = 1e-12
Q = 784
D = 512
K = 16384
KT = 2048
NT = K // KT


def _knn_kernel(q_ref, m_ref, minval_ref, sidx_ref, sstar_ref,
                qs_ref, ming_ref):
    t = pl.program_id(0)
    nt = pl.num_programs(0)

    @pl.when(t == 0)
    def _prescale():
        qs_ref[...] = -2.0 * q_ref[...]

    m = m_ref[...]
    qm = jax.lax.dot_general(
        qs_ref[...], m, (((1,), (1,)), ((), ())),
        preferred_element_type=jnp.float32)
    m2 = jnp.sum(m * m, axis=1)
    g = qm + m2[None, :]
    rowmin = jnp.min(g, axis=1, keepdims=True)

    @pl.when(t == 0)
    def _init():
        ming_ref[...] = rowmin

    @pl.when(t > 0)
    def _update():
        ming_ref[...] = jnp.minimum(rowmin, ming_ref[...])

    @pl.when(t == nt - 1)
    def _finalize():
        q = q_ref[...]
        q2 = jnp.sum(q * q, axis=1, keepdims=True)
        mv2 = jnp.maximum(ming_ref[...] + q2, 0.0)
        minval_ref[...] = jnp.sqrt(mv2 + EPS)
        smax = jnp.max(mv2)
        rows = jax.lax.broadcasted_iota(jnp.int32, mv2.shape, 0)
        sidx_ref[0, 0] = jnp.min(jnp.where(mv2 == smax, rows, Q))
        sstar_ref[0, 0] = jnp.sqrt(smax + EPS)


def _reweight_kernel(m_ref, mt_ref, sstar_ref, s_ref,
                     allm_ref, m2s_ref, wt_ref, ws_ref):
    t = pl.program_id(0)
    nt = pl.num_programs(0)
    m = m_ref[...]
    allm_ref[pl.ds(t * KT, KT), :] = m
    m2 = jax.lax.dot_general(
        jnp.ones((1, D), jnp.float32), m * m, (((1,), (1,)), ((), ())),
        preferred_element_type=jnp.float32)
    ct = jax.lax.dot_general(
        mt_ref[...], m, (((1,), (1,)), ((), ())),
        preferred_element_type=jnp.float32)
    m2s_ref[pl.ds(t, 1), :] = m2
    wt_ref[pl.ds(t, 1), :] = m2 - 2.0 * ct

    @pl.when(t == nt - 1)
    def _finalize():
        mt = mt_ref[...]
        t2 = jnp.sum(mt * mt)
        wt = jnp.maximum(wt_ref[...] + t2, 0.0)
        rows = jax.lax.broadcasted_iota(jnp.int32, wt.shape, 0)
        lanes = jax.lax.broadcasted_iota(jnp.int32, wt.shape, 1)
        flat = rows * KT + lanes
        star = jnp.min(jnp.where(wt == jnp.min(wt), flat, K))
        mstar = allm_ref[pl.ds(star, 1), :]
        s2 = jnp.sum(mstar * mstar)
        for j in range(NT):
            cs = jax.lax.dot_general(
                mstar, allm_ref[pl.ds(j * KT, KT), :],
                (((1,), (1,)), ((), ())),
                preferred_element_type=jnp.float32)
            ws_ref[pl.ds(j, 1), :] = m2s_ref[pl.ds(j, 1), :] - 2.0 * cs
        ws = jnp.maximum(ws_ref[...] + s2, 0.0)
        acc = 0.0
        for _ in range(3):
            mn = jnp.min(ws)
            idx = jnp.min(jnp.where(ws == mn, flat, K))
            dj2 = jnp.min(jnp.where(flat == idx, wt, jnp.inf))
            acc = acc + jnp.exp(jnp.sqrt(dj2 + EPS))
            ws = jnp.where(flat == idx, jnp.inf, ws)
        s_star = sstar_ref[0, 0]
        s_ref[0, 0] = (1.0 - jnp.exp(s_star) / acc) * s_star


@functools.partial(jax.jit, static_argnums=())
def kernel(queries, memory):
    minval, sidx, sstar = pl.pallas_call(
        _knn_kernel,
        grid=(NT,),
        in_specs=[
            pl.BlockSpec((Q, D), lambda t: (0, 0)),
            pl.BlockSpec((KT, D), lambda t: (t, 0)),
        ],
        out_specs=[
            pl.BlockSpec((Q, 1), lambda t: (0, 0)),
            pl.BlockSpec(memory_space=pltpu.SMEM),
            pl.BlockSpec(memory_space=pltpu.SMEM),
        ],
        out_shape=[
            jax.ShapeDtypeStruct((Q, 1), jnp.float32),
            jax.ShapeDtypeStruct((1, 1), jnp.int32),
            jax.ShapeDtypeStruct((1, 1), jnp.float32),
        ],
        scratch_shapes=[
            pltpu.VMEM((Q, D), jnp.float32),
            pltpu.VMEM((Q, 1), jnp.float32),
        ],
    )(queries, memory)

    m_test = jnp.take(queries, sidx[0, 0], axis=0)[None, :]

    s = pl.pallas_call(
        _reweight_kernel,
        grid=(NT,),
        in_specs=[
            pl.BlockSpec((KT, D), lambda t: (t, 0)),
            pl.BlockSpec((1, D), lambda t: (0, 0)),
            pl.BlockSpec(memory_space=pltpu.SMEM),
        ],
        out_specs=pl.BlockSpec(memory_space=pltpu.SMEM),
        out_shape=jax.ShapeDtypeStruct((1, 1), jnp.float32),
        scratch_shapes=[
            pltpu.VMEM((K, D), jnp.float32),
            pltpu.VMEM((NT, KT), jnp.float32),
            pltpu.VMEM((NT, KT), jnp.float32),
            pltpu.VMEM((NT, KT), jnp.float32),
        ],
    )(memory, m_test, sstar)

    return (s[0, 0], minval.reshape(Q))


# speedup vs baseline: 1.1954x; 1.0397x over previous
import functools

import jax
import jax.numpy as jnp
from jax.experimental import pallas as pl
from jax.experimental.pallas import tpu as pltpu

EPS = 1e-12
Q = 784
D = 512
K = 16384
KT = 1024
NT = K // KT


def _patchcore_kernel(q_ref, m_ref, minval_ref, s_ref,
                      allm_ref, m2h_ref, ming_ref):
    t = pl.program_id(0)
    nt = pl.num_programs(0)

    m = m_ref[...]
    allm_ref[pl.ds(t * KT, KT), :] = m
    qm = jax.lax.dot_general(
        q_ref[...], m, (((1,), (1,)), ((), ())),
        preferred_element_type=jnp.float32)
    m2h = 0.5 * jax.lax.dot_general(
        jnp.ones((1, D), jnp.float32), m * m, (((1,), (1,)), ((), ())),
        preferred_element_type=jnp.float32)
    m2h_ref[0:1, pl.ds(t * KT, KT)] = m2h
    h = m2h - qm
    rowmin = jnp.min(h, axis=1, keepdims=True)

    @pl.when(t == 0)
    def _init():
        ming_ref[...] = rowmin

    @pl.when(t > 0)
    def _update():
        ming_ref[...] = jnp.minimum(rowmin, ming_ref[...])

    @pl.when(t == nt - 1)
    def _finalize():
        q = q_ref[...]
        q2 = jnp.sum(q * q, axis=1, keepdims=True)
        mv2 = jnp.maximum(2.0 * ming_ref[...] + q2, 0.0)
        minval_ref[...] = jnp.sqrt(mv2 + EPS)
        smax = jnp.max(mv2)
        rows = jax.lax.broadcasted_iota(jnp.int32, mv2.shape, 0)
        sidx = jnp.min(jnp.where(mv2 == smax, rows, Q))
        s_star = jnp.sqrt(smax + EPS)

        mt = q_ref[pl.ds(sidx, 1), :]
        t2 = jnp.sum(mt * mt)
        m2h_row = m2h_ref[...]
        ct = jax.lax.dot_general(
            mt, allm_ref[...], (((1,), (1,)), ((), ())),
            preferred_element_type=jnp.float32)
        wt = jnp.maximum(2.0 * (m2h_row - ct) + t2, 0.0)
        flat = jax.lax.broadcasted_iota(jnp.int32, wt.shape, 1)
        star = jnp.min(jnp.where(wt == jnp.min(wt), flat, K))
        mstar = allm_ref[pl.ds(star, 1), :]
        s2 = jnp.sum(mstar * mstar)
        cs = jax.lax.dot_general(
            mstar, allm_ref[...], (((1,), (1,)), ((), ())),
            preferred_element_type=jnp.float32)
        ws = jnp.maximum(2.0 * (m2h_row - cs) + s2, 0.0)
        acc = 0.0
        for _ in range(3):
            mn = jnp.min(ws)
            idx = jnp.min(jnp.where(ws == mn, flat, K))
            dj2 = jnp.min(jnp.where(flat == idx, wt, jnp.inf))
            acc = acc + jnp.exp(jnp.sqrt(dj2 + EPS))
            ws = jnp.where(flat == idx, jnp.inf, ws)
        s_ref[0, 0] = (1.0 - jnp.exp(s_star) / acc) * s_star


@functools.partial(jax.jit, static_argnums=())
def kernel(queries, memory):
    minval, s = pl.pallas_call(
        _patchcore_kernel,
        grid=(NT,),
        in_specs=[
            pl.BlockSpec((Q, D), lambda t: (0, 0)),
            pl.BlockSpec((KT, D), lambda t: (t, 0)),
        ],
        out_specs=[
            pl.BlockSpec((Q, 1), lambda t: (0, 0)),
            pl.BlockSpec(memory_space=pltpu.SMEM),
        ],
        out_shape=[
            jax.ShapeDtypeStruct((Q, 1), jnp.float32),
            jax.ShapeDtypeStruct((1, 1), jnp.float32),
        ],
        scratch_shapes=[
            pltpu.VMEM((K, D), jnp.float32),
            pltpu.VMEM((1, K), jnp.float32),
            pltpu.VMEM((Q, 1), jnp.float32),
        ],
    )(queries, memory)

    return (s[0, 0], minval.reshape(Q))
